# Initial kernel scaffold; baseline (speedup 1.0000x reference)
import jax
import jax.numpy as jnp
from jax.experimental import pallas as pl


def kernel(input_ids, word_emb, pos_emb, tt_emb, gamma, beta):
    raise NotImplementedError("write your pallas kernel here")


# speedup vs baseline: 2.0763x; 2.0763x over previous
import functools

import jax
import jax.numpy as jnp
from jax import lax
from jax.experimental import pallas as pl
from jax.experimental.pallas import tpu as pltpu
from jax.experimental.pallas import tpu_sc as plsc

PAD = 1
EPS = 1e-12
LANES = 16
CHUNK = 128


def _pos_body(ids_ref, pid_ref):
    x = ids_ref[...]
    m = (x != PAD).astype(jnp.int32)
    inc = m
    seq = x.shape[1]
    k = 1
    while k < seq:
        shifted = jnp.concatenate(
            [jnp.zeros((x.shape[0], k), jnp.int32), inc[:, :-k]], axis=1)
        inc = inc + shifted
        k *= 2
    pid_ref[...] = inc * m + PAD


def _emb_body(ids_hbm, pid_hbm, word_hbm, pos_hbm, tt_hbm, g_hbm, b_hbm,
              out_hbm,
              ids_v, pid_v, wbuf0, wbuf1, pbuf0, pbuf1, obuf0, obuf1,
              ttv, gv, bv,
              sem_w0, sem_w1, sem_p0, sem_p1, sem_o0, sem_o1):
    num_cores = 2
    wid = lax.axis_index("s") * num_cores + lax.axis_index("c")

    n_chunks = ids_v.shape[0]

    pltpu.sync_copy(ids_hbm.at[wid], ids_v)
    pltpu.sync_copy(pid_hbm.at[wid], pid_v)
    pltpu.sync_copy(tt_hbm.at[0], ttv)
    pltpu.sync_copy(g_hbm, gv)
    pltpu.sync_copy(b_hbm, bv)

    wbufs = [wbuf0, wbuf1]
    pbufs = [pbuf0, pbuf1]
    obufs = [obuf0, obuf1]
    sems_w = [sem_w0, sem_w1]
    sems_p = [sem_p0, sem_p1]
    sems_o = [sem_o0, sem_o1]

    def fire(c):
        par = c % 2
        cw = pltpu.async_copy(word_hbm.at[ids_v.at[c]], wbufs[par], sems_w[par])
        cp = pltpu.async_copy(pos_hbm.at[pid_v.at[c]], pbufs[par], sems_p[par])
        return cw, cp

    inv_h = 1.0 / (8 * LANES)

    def compute(c):
        par = c % 2
        bw = wbufs[par]
        bp = pbufs[par]
        bo = obufs[par]

        def tok(t, _):
            svec = jnp.zeros((LANES,), jnp.float32)
            qvec = jnp.zeros((LANES,), jnp.float32)
            for j in range(8):
                sl = pl.ds(j * LANES, LANES)
                x = bw[t, sl] + bp[t, sl] + ttv[sl]
                bo[t, sl] = x
                svec = svec + x
                qvec = qvec + x * x
            lane = lax.iota(jnp.int32, LANES)

            def allsum(v):
                for k in (1, 2, 4, 8):
                    v = v + v.at[lane ^ k].get(mode="promise_in_bounds")
                return v

            total = allsum(svec)
            totalq = allsum(qvec)
            mv = total * inv_h
            var = totalq * inv_h - mv * mv
            v = var + EPS
            vi = lax.bitcast_convert_type(v, jnp.int32)
            y = lax.bitcast_convert_type(
                jnp.int32(0x5F3759DF) - (vi >> 1), jnp.float32)
            for _ in range(3):
                y = y * (1.5 - 0.5 * v * y * y)
            for j in range(8):
                sl = pl.ds(j * LANES, LANES)
                bo[t, sl] = (bo[t, sl] - mv) * y * gv[sl] + bv[sl]
            return 0

        lax.fori_loop(0, CHUNK, tok, 0)

    copies = [None] * n_chunks
    out_copies = [None, None]
    copies[0] = fire(0)
    for c in range(n_chunks):
        if c + 1 < n_chunks:
            copies[c + 1] = fire(c + 1)
        cw, cp = copies[c]
        cw.wait()
        cp.wait()
        if out_copies[c % 2] is not None:
            out_copies[c % 2].wait()
        compute(c)
        oc = pltpu.async_copy(obufs[c % 2],
                              out_hbm.at[wid, pl.ds(c * CHUNK, CHUNK)],
                              sems_o[c % 2])
        out_copies[c % 2] = oc
    for oc in out_copies:
        if oc is not None:
            oc.wait()


def kernel(input_ids, word_emb, pos_emb, tt_emb, gamma, beta):
    B, S = input_ids.shape
    V, H = word_emb.shape
    assert H == 128 and S % CHUNK == 0 and B == 32
    n_chunks = S // CHUNK

    ids = input_ids.astype(jnp.int32)
    pid = pl.pallas_call(
        _pos_body,
        out_shape=jax.ShapeDtypeStruct((B, S), jnp.int32),
    )(ids)

    ids3 = ids.reshape(B, n_chunks, CHUNK)
    pid3 = pid.reshape(B, n_chunks, CHUNK)

    mesh = plsc.VectorSubcoreMesh(core_axis_name="c", subcore_axis_name="s")
    f = functools.partial(
        pl.kernel,
        mesh=mesh,
        out_type=jax.ShapeDtypeStruct((B, S, H), jnp.float32),
        scratch_types=[
            pltpu.VMEM((n_chunks, CHUNK), jnp.int32),
            pltpu.VMEM((n_chunks, CHUNK), jnp.int32),
            pltpu.VMEM((CHUNK, H), jnp.float32),
            pltpu.VMEM((CHUNK, H), jnp.float32),
            pltpu.VMEM((CHUNK, H), jnp.float32),
            pltpu.VMEM((CHUNK, H), jnp.float32),
            pltpu.VMEM((CHUNK, H), jnp.float32),
            pltpu.VMEM((CHUNK, H), jnp.float32),
            pltpu.VMEM((H,), jnp.float32),
            pltpu.VMEM((H,), jnp.float32),
            pltpu.VMEM((H,), jnp.float32),
            pltpu.SemaphoreType.DMA,
            pltpu.SemaphoreType.DMA,
            pltpu.SemaphoreType.DMA,
            pltpu.SemaphoreType.DMA,
            pltpu.SemaphoreType.DMA,
            pltpu.SemaphoreType.DMA,
        ],
    )(_emb_body)
    return f(ids3, pid3, word_emb, pos_emb, tt_emb, gamma, beta)
